# baseline (speedup 1.0000x reference)
import jax
import jax.numpy as jnp
from jax.experimental import pallas as pl
from jax.experimental.pallas import tpu as pltpu

H = 2048
E = 64
TOP_K = 2
N_TOK = 8192
BT = 1024
NBLK = N_TOK // BT


def _main_kernel(x_ref, wc_ref, b1_ref, w2_ref, b2_ref, ebias_ref,
                 ts_ref, ti_ref, cnt_ref, ps_ref, cs_ref):
    x = x_ref[...]
    both = jnp.dot(x, wc_ref[...], preferred_element_type=jnp.float32)
    both_t = both.T
    logits = both_t[:E]
    h1pre = both_t[E:]

    m = jnp.max(logits, axis=0, keepdims=True)
    ex = jnp.exp(logits - m)
    scores = ex / jnp.sum(ex, axis=0, keepdims=True)

    biased = scores + ebias_ref[...]
    iota = jax.lax.broadcasted_iota(jnp.int32, (E, BT), 0)
    m1 = jnp.max(biased, axis=0, keepdims=True)
    sel1 = iota == jnp.min(jnp.where(biased == m1, iota, E),
                           axis=0, keepdims=True)
    masked = jnp.where(sel1, -jnp.inf, biased)
    m2 = jnp.max(masked, axis=0, keepdims=True)
    sel2 = iota == jnp.min(jnp.where(masked == m2, iota, E),
                           axis=0, keepdims=True)

    s1 = jnp.sum(jnp.where(sel1, scores, 0.0), axis=0, keepdims=True)
    s2 = jnp.sum(jnp.where(sel2, scores, 0.0), axis=0, keepdims=True)
    rden = 1.0 / (s1 + s2)
    ts_ref[...] = jnp.concatenate([s1 * rden, s2 * rden], axis=0)
    ti_ref[...] = jnp.concatenate(
        [jnp.sum(jnp.where(sel1, iota, 0), axis=0, keepdims=True),
         jnp.sum(jnp.where(sel2, iota, 0), axis=0, keepdims=True)], axis=0)

    cnt_ref[0] = jnp.sum(sel1.astype(jnp.float32) + sel2.astype(jnp.float32),
                         axis=1, keepdims=True)
    ps_ref[0] = jnp.sum(scores, axis=1, keepdims=True)

    h1 = jnp.maximum(h1pre + b1_ref[...], 0.0)
    c = jax.nn.sigmoid(jnp.sum(h1 * w2_ref[...], axis=0, keepdims=True)
                       + b2_ref[...])
    cs_ref[...] = jnp.sum(c).reshape(1, 1, 1)


def _finalize_kernel(ts2_ref, ti2_ref, cnt_ref, ps_ref, cs_ref,
                     ts_ref, ti_ref, loss_ref):
    ts_ref[...] = ts2_ref[...].T
    ti_ref[...] = ti2_ref[...].T
    counts = jnp.sum(cnt_ref[...], axis=0)
    psum = jnp.sum(ps_ref[...], axis=0)
    csum = jnp.sum(cs_ref[...])
    aux = E * jnp.sum(counts * psum) / (N_TOK * TOP_K * N_TOK)
    loss_ref[...] = (aux * (0.5 + csum / N_TOK)).reshape(1, 1)


@jax.jit
def kernel(hidden_states, Wg, W1, b1, W2, b2, expert_bias):
    x = hidden_states.reshape(-1, H)
    wc = jnp.concatenate([Wg, W1], axis=0).T
    b1r = b1.reshape(E, 1)
    w2r = W2.reshape(E, 1)
    b2r = b2.reshape(1, 1)
    ebr = expert_bias.reshape(E, 1)

    ts2, ti2, cnt, ps, cs = pl.pallas_call(
        _main_kernel,
        grid=(NBLK,),
        in_specs=[
            pl.BlockSpec((BT, H), lambda i: (i, 0)),
            pl.BlockSpec((H, 2 * E), lambda i: (0, 0)),
            pl.BlockSpec((E, 1), lambda i: (0, 0)),
            pl.BlockSpec((E, 1), lambda i: (0, 0)),
            pl.BlockSpec((1, 1), lambda i: (0, 0)),
            pl.BlockSpec((E, 1), lambda i: (0, 0)),
        ],
        out_specs=[
            pl.BlockSpec((TOP_K, BT), lambda i: (0, i)),
            pl.BlockSpec((TOP_K, BT), lambda i: (0, i)),
            pl.BlockSpec((1, E, 1), lambda i: (i, 0, 0)),
            pl.BlockSpec((1, E, 1), lambda i: (i, 0, 0)),
            pl.BlockSpec((1, 1, 1), lambda i: (i, 0, 0)),
        ],
        out_shape=[
            jax.ShapeDtypeStruct((TOP_K, N_TOK), jnp.float32),
            jax.ShapeDtypeStruct((TOP_K, N_TOK), jnp.int32),
            jax.ShapeDtypeStruct((NBLK, E, 1), jnp.float32),
            jax.ShapeDtypeStruct((NBLK, E, 1), jnp.float32),
            jax.ShapeDtypeStruct((NBLK, 1, 1), jnp.float32),
        ],
        compiler_params=pltpu.CompilerParams(
            dimension_semantics=("parallel",)),
    )(x, wc, b1r, w2r, b2r, ebr)

    ts, ti, loss = pl.pallas_call(
        _finalize_kernel,
        out_shape=[
            jax.ShapeDtypeStruct((N_TOK, TOP_K), jnp.float32),
            jax.ShapeDtypeStruct((N_TOK, TOP_K), jnp.int32),
            jax.ShapeDtypeStruct((1, 1), jnp.float32),
        ],
    )(ts2, ti2, cnt, ps, cs)

    return ts, ti, loss.reshape(())


# speedup vs baseline: 1.0378x; 1.0378x over previous
import jax
import jax.numpy as jnp
from jax.experimental import pallas as pl
from jax.experimental.pallas import tpu as pltpu

H = 2048
E = 64
TOP_K = 2
N_TOK = 8192
BT = 1024
NBLK = N_TOK // BT


def _main_kernel(x_ref, wc_ref, b1_ref, w2_ref, b2_ref, ebias_ref,
                 ts_ref, ti_ref, cnt_ref, ps_ref, cs_ref):
    x = x_ref[...]
    both = jnp.dot(x, wc_ref[...], preferred_element_type=jnp.float32)
    both_t = both.T
    logits = both_t[:E]
    h1pre = both_t[E:]

    m = jnp.max(logits, axis=0, keepdims=True)
    ex = jnp.exp(logits - m)
    scores = ex / jnp.sum(ex, axis=0, keepdims=True)

    biased = scores + ebias_ref[...]
    iota = jax.lax.broadcasted_iota(jnp.int32, (E, BT), 0)
    m1 = jnp.max(biased, axis=0, keepdims=True)
    sel1 = iota == jnp.min(jnp.where(biased == m1, iota, E),
                           axis=0, keepdims=True)
    masked = jnp.where(sel1, -jnp.inf, biased)
    m2 = jnp.max(masked, axis=0, keepdims=True)
    sel2 = iota == jnp.min(jnp.where(masked == m2, iota, E),
                           axis=0, keepdims=True)

    s1 = jnp.sum(jnp.where(sel1, scores, 0.0), axis=0, keepdims=True)
    s2 = jnp.sum(jnp.where(sel2, scores, 0.0), axis=0, keepdims=True)
    rden = 1.0 / (s1 + s2)
    ts_ref[...] = jnp.concatenate([s1 * rden, s2 * rden], axis=0).T
    ti_ref[...] = jnp.concatenate(
        [jnp.sum(jnp.where(sel1, iota, 0), axis=0, keepdims=True),
         jnp.sum(jnp.where(sel2, iota, 0), axis=0, keepdims=True)], axis=0).T

    cnt_ref[0] = jnp.sum(sel1.astype(jnp.float32) + sel2.astype(jnp.float32),
                         axis=1, keepdims=True)
    ps_ref[0] = jnp.sum(scores, axis=1, keepdims=True)

    h1 = jnp.maximum(h1pre + b1_ref[...], 0.0)
    c = jax.nn.sigmoid(jnp.sum(h1 * w2_ref[...], axis=0, keepdims=True)
                       + b2_ref[...])
    cs_ref[...] = jnp.sum(c).reshape(1, 1, 1)


def _finalize_kernel(cnt_ref, ps_ref, cs_ref, loss_ref):
    counts = jnp.sum(cnt_ref[...], axis=0)
    psum = jnp.sum(ps_ref[...], axis=0)
    csum = jnp.sum(cs_ref[...])
    aux = E * jnp.sum(counts * psum) / (N_TOK * TOP_K * N_TOK)
    loss_ref[...] = (aux * (0.5 + csum / N_TOK)).reshape(1, 1)


@jax.jit
def kernel(hidden_states, Wg, W1, b1, W2, b2, expert_bias):
    x = hidden_states.reshape(-1, H)
    wc = jnp.concatenate([Wg, W1], axis=0).T
    b1r = b1.reshape(E, 1)
    w2r = W2.reshape(E, 1)
    b2r = b2.reshape(1, 1)
    ebr = expert_bias.reshape(E, 1)

    ts, ti, cnt, ps, cs = pl.pallas_call(
        _main_kernel,
        grid=(NBLK,),
        in_specs=[
            pl.BlockSpec((BT, H), lambda i: (i, 0)),
            pl.BlockSpec((H, 2 * E), lambda i: (0, 0)),
            pl.BlockSpec((E, 1), lambda i: (0, 0)),
            pl.BlockSpec((E, 1), lambda i: (0, 0)),
            pl.BlockSpec((1, 1), lambda i: (0, 0)),
            pl.BlockSpec((E, 1), lambda i: (0, 0)),
        ],
        out_specs=[
            pl.BlockSpec((BT, TOP_K), lambda i: (i, 0)),
            pl.BlockSpec((BT, TOP_K), lambda i: (i, 0)),
            pl.BlockSpec((1, E, 1), lambda i: (i, 0, 0)),
            pl.BlockSpec((1, E, 1), lambda i: (i, 0, 0)),
            pl.BlockSpec((1, 1, 1), lambda i: (i, 0, 0)),
        ],
        out_shape=[
            jax.ShapeDtypeStruct((N_TOK, TOP_K), jnp.float32),
            jax.ShapeDtypeStruct((N_TOK, TOP_K), jnp.int32),
            jax.ShapeDtypeStruct((NBLK, E, 1), jnp.float32),
            jax.ShapeDtypeStruct((NBLK, E, 1), jnp.float32),
            jax.ShapeDtypeStruct((NBLK, 1, 1), jnp.float32),
        ],
        compiler_params=pltpu.CompilerParams(
            dimension_semantics=("parallel",)),
    )(x, wc, b1r, w2r, b2r, ebr)

    loss = pl.pallas_call(
        _finalize_kernel,
        out_shape=jax.ShapeDtypeStruct((1, 1), jnp.float32),
    )(cnt, ps, cs)

    return ts, ti, loss.reshape(())


# speedup vs baseline: 1.3125x; 1.2646x over previous
import jax
import jax.numpy as jnp
from jax.experimental import pallas as pl
from jax.experimental.pallas import tpu as pltpu

H = 2048
E = 64
TOP_K = 2
N_TOK = 8192
BT = 1024
NBLK = N_TOK // BT


def _main_kernel(x_ref, wc_ref, b1_ref, w2_ref, b2_ref, ebias_ref,
                 ts_ref, ti_ref, loss_ref, cnt_acc, ps_acc, cs_acc):
    i = pl.program_id(0)
    x = x_ref[...]
    both = jnp.dot(x, wc_ref[...], preferred_element_type=jnp.float32)
    both_t = both.T
    logits = both_t[:E]
    h1pre = both_t[E:]

    m = jnp.max(logits, axis=0, keepdims=True)
    ex = jnp.exp(logits - m)
    scores = ex / jnp.sum(ex, axis=0, keepdims=True)

    biased = scores + ebias_ref[...]
    iota = jax.lax.broadcasted_iota(jnp.int32, (E, BT), 0)
    m1 = jnp.max(biased, axis=0, keepdims=True)
    sel1 = iota == jnp.min(jnp.where(biased == m1, iota, E),
                           axis=0, keepdims=True)
    masked = jnp.where(sel1, -jnp.inf, biased)
    m2 = jnp.max(masked, axis=0, keepdims=True)
    sel2 = iota == jnp.min(jnp.where(masked == m2, iota, E),
                           axis=0, keepdims=True)

    s1 = jnp.sum(jnp.where(sel1, scores, 0.0), axis=0, keepdims=True)
    s2 = jnp.sum(jnp.where(sel2, scores, 0.0), axis=0, keepdims=True)
    rden = 1.0 / (s1 + s2)
    ts_ref[...] = jnp.concatenate([s1 * rden, s2 * rden], axis=0)
    ti_ref[...] = jnp.concatenate(
        [jnp.sum(jnp.where(sel1, iota, 0), axis=0, keepdims=True),
         jnp.sum(jnp.where(sel2, iota, 0), axis=0, keepdims=True)], axis=0)

    h1 = jnp.maximum(h1pre + b1_ref[...], 0.0)
    c = jax.nn.sigmoid(jnp.sum(h1 * w2_ref[...], axis=0, keepdims=True)
                       + b2_ref[...])

    cnt_blk = jnp.sum(sel1.astype(jnp.float32) + sel2.astype(jnp.float32),
                      axis=1, keepdims=True)
    ps_blk = jnp.sum(scores, axis=1, keepdims=True)
    cs_blk = jnp.sum(c).reshape(1, 1)

    @pl.when(i == 0)
    def _init():
        cnt_acc[...] = cnt_blk
        ps_acc[...] = ps_blk
        cs_acc[...] = cs_blk

    @pl.when(i > 0)
    def _acc():
        cnt_acc[...] += cnt_blk
        ps_acc[...] += ps_blk
        cs_acc[...] += cs_blk

    @pl.when(i == NBLK - 1)
    def _fin():
        aux = E * jnp.sum(cnt_acc[...] * ps_acc[...]) / (N_TOK * TOP_K * N_TOK)
        loss_ref[...] = aux * (0.5 + cs_acc[...] / N_TOK)


@jax.jit
def kernel(hidden_states, Wg, W1, b1, W2, b2, expert_bias):
    x = hidden_states.reshape(-1, H)
    wc = jnp.concatenate([Wg, W1], axis=0).T
    b1r = b1.reshape(E, 1)
    w2r = W2.reshape(E, 1)
    b2r = b2.reshape(1, 1)
    ebr = expert_bias.reshape(E, 1)

    ts, ti, loss = pl.pallas_call(
        _main_kernel,
        grid=(NBLK,),
        in_specs=[
            pl.BlockSpec((BT, H), lambda i: (i, 0)),
            pl.BlockSpec((H, 2 * E), lambda i: (0, 0)),
            pl.BlockSpec((E, 1), lambda i: (0, 0)),
            pl.BlockSpec((E, 1), lambda i: (0, 0)),
            pl.BlockSpec((1, 1), lambda i: (0, 0)),
            pl.BlockSpec((E, 1), lambda i: (0, 0)),
        ],
        out_specs=[
            pl.BlockSpec((TOP_K, BT), lambda i: (0, i)),
            pl.BlockSpec((TOP_K, BT), lambda i: (0, i)),
            pl.BlockSpec((1, 1), lambda i: (0, 0)),
        ],
        out_shape=[
            jax.ShapeDtypeStruct((TOP_K, N_TOK), jnp.float32),
            jax.ShapeDtypeStruct((TOP_K, N_TOK), jnp.int32),
            jax.ShapeDtypeStruct((1, 1), jnp.float32),
        ],
        scratch_shapes=[
            pltpu.VMEM((E, 1), jnp.float32),
            pltpu.VMEM((E, 1), jnp.float32),
            pltpu.VMEM((1, 1), jnp.float32),
        ],
        compiler_params=pltpu.CompilerParams(
            dimension_semantics=("arbitrary",)),
    )(x, wc, b1r, w2r, b2r, ebr)

    return ts.T, ti.T, loss.reshape(())
